# baseline (speedup 1.0000x reference)
import numpy as np

import jax
import jax.numpy as jnp
from jax.experimental import pallas as pl
from jax.experimental.shard_map import shard_map
from jax.sharding import Mesh, PartitionSpec as P


B, N, M, K = 32, 2048, 512, 8
BOTTLENECK = 1024


def _mlp_kernel(gf_ref, w1_ref, b1_ref, g1_ref, be1_ref,
                w2_ref, b2_ref, g2_ref, be2_ref,
                w3_ref, b3_ref, g3_ref, be3_ref,
                w4_ref, b4_ref, y_ref):
    def bn_relu(y, g, be):
        mean = jnp.mean(y, axis=0, keepdims=True)
        var = jnp.mean((y - mean) * (y - mean), axis=0, keepdims=True)
        return jax.nn.relu((y - mean) * jax.lax.rsqrt(var + 1e-5) * g + be)

    y = jnp.dot(gf_ref[...], w1_ref[...], preferred_element_type=jnp.float32)
    y = bn_relu(y + b1_ref[...], g1_ref[...], be1_ref[...])
    y = jnp.dot(y, w2_ref[...], preferred_element_type=jnp.float32)
    y = bn_relu(y + b2_ref[...], g2_ref[...], be2_ref[...])
    y = jnp.dot(y, w3_ref[...], preferred_element_type=jnp.float32)
    y = bn_relu(y + b3_ref[...], g3_ref[...], be3_ref[...])
    y = jnp.dot(y, w4_ref[...], preferred_element_type=jnp.float32)
    y_ref[...] = y + b4_ref[...]


def _proj_kernel(q_ref, xt_ref, isig_ref, out_ref):
    q = q_ref[0]
    pt = xt_ref[0]
    inv_sigma = isig_ref[0, 0]

    qp2 = jnp.dot(-2.0 * q, pt, preferred_element_type=jnp.float32)
    q2 = jnp.sum(q * q, axis=1, keepdims=True)
    p2 = jnp.sum(pt * pt, axis=0, keepdims=True)
    d2sel = (q2 + qp2) + p2

    dmin = jnp.min(d2sel, axis=1, keepdims=True)
    t = dmin
    for _ in range(K - 1):
        t = jnp.min(jnp.where(d2sel <= t, jnp.inf, d2sel), axis=1,
                    keepdims=True)

    c2 = inv_sigma * 1.4426950408889634
    w = jnp.where(d2sel <= t, jnp.exp2((dmin - d2sel) * c2), 0.0)

    px = pt[0:1, :]
    py = pt[1:2, :]
    pz = pt[2:3, :]
    wsum = jnp.sum(w, axis=1, keepdims=True)
    ox = jnp.sum(w * px, axis=1, keepdims=True)
    oy = jnp.sum(w * py, axis=1, keepdims=True)
    oz = jnp.sum(w * pz, axis=1, keepdims=True)
    out_ref[0] = jnp.concatenate([ox, oy, oz], axis=1) / wsum


def _run_mlp(global_feat, W1t, b1, g1, be1, W2t, b2, g2, be2, W3t, b3, g3,
             be3, W4t, b4):
    return pl.pallas_call(
        _mlp_kernel,
        out_shape=jax.ShapeDtypeStruct((B, 3 * M), jnp.float32),
    )(global_feat, W1t, b1, g1, be1, W2t, b2, g2, be2, W3t, b3, g3, be3,
      W4t, b4)


def _run_proj(generated, xt, inv_sigma, nb):
    return pl.pallas_call(
        _proj_kernel,
        grid=(nb,),
        in_specs=[
            pl.BlockSpec((1, M, 3), lambda b: (b, 0, 0)),
            pl.BlockSpec((1, 3, N), lambda b: (b, 0, 0)),
            pl.BlockSpec((1, 1), lambda b: (0, 0)),
        ],
        out_specs=pl.BlockSpec((1, M, 3), lambda b: (b, 0, 0)),
        out_shape=jax.ShapeDtypeStruct((nb, M, 3), jnp.float32),
    )(generated, xt, inv_sigma)


def kernel(x, global_feat, W1, b1, g1, be1, W2, b2, g2, be2, W3, b3, g3, be3,
           W4, b4, temperature):
    f32 = jnp.float32
    sigma = jnp.maximum(temperature * temperature, 0.01)
    inv_sigma = (1.0 / sigma).reshape(1, 1).astype(f32)
    mlp_args = (global_feat, W1.T, b1.reshape(1, -1), g1.reshape(1, -1),
                be1.reshape(1, -1), W2.T, b2.reshape(1, -1),
                g2.reshape(1, -1), be2.reshape(1, -1), W3.T,
                b3.reshape(1, -1), g3.reshape(1, -1), be3.reshape(1, -1),
                W4.T, b4.reshape(1, -1))

    devs = jax.devices()
    nd = 2 if (len(devs) >= 2 and B % 2 == 0) else 1
    if nd == 1:
        y = _run_mlp(*mlp_args)
        generated = jnp.transpose(y.reshape(B, 3, M), (0, 2, 1))
        proj = _run_proj(generated, jnp.transpose(x, (0, 2, 1)), inv_sigma, B)
        return generated, proj

    nb = B // nd
    mesh = Mesh(np.asarray(devs[:nd]), ("d",))

    def shard_fn(x_s, isig, *margs):
        y = _run_mlp(*margs)
        gen_full = jnp.transpose(y.reshape(B, 3, M), (0, 2, 1))
        i = jax.lax.axis_index("d")
        gen_s = jax.lax.dynamic_slice_in_dim(gen_full, i * nb, nb, 0)
        proj_s = _run_proj(gen_s, jnp.transpose(x_s, (0, 2, 1)), isig, nb)
        return gen_s, proj_s

    fn = shard_map(
        shard_fn, mesh=mesh,
        in_specs=(P("d"),) + (P(),) * (1 + len(mlp_args)),
        out_specs=(P("d"), P("d")),
        check_rep=False,
    )
    return fn(x, inv_sigma, *mlp_args)


# speedup vs baseline: 1.8474x; 1.8474x over previous
import jax
import jax.numpy as jnp
from jax.experimental import pallas as pl


B, N, M, K = 32, 2048, 512, 8
BOTTLENECK = 1024


def _mlp_kernel(gf_ref, w1_ref, b1_ref, g1_ref, be1_ref,
                w2_ref, b2_ref, g2_ref, be2_ref,
                w3_ref, b3_ref, g3_ref, be3_ref,
                w4_ref, b4_ref, y_ref):
    def bn_relu(y, g, be):
        mean = jnp.mean(y, axis=0, keepdims=True)
        var = jnp.mean((y - mean) * (y - mean), axis=0, keepdims=True)
        return jax.nn.relu((y - mean) * jax.lax.rsqrt(var + 1e-5) * g + be)

    y = jnp.dot(gf_ref[...], w1_ref[...], preferred_element_type=jnp.float32)
    y = bn_relu(y + b1_ref[...], g1_ref[...], be1_ref[...])
    y = jnp.dot(y, w2_ref[...], preferred_element_type=jnp.float32)
    y = bn_relu(y + b2_ref[...], g2_ref[...], be2_ref[...])
    y = jnp.dot(y, w3_ref[...], preferred_element_type=jnp.float32)
    y = bn_relu(y + b3_ref[...], g3_ref[...], be3_ref[...])
    y = jnp.dot(y, w4_ref[...], preferred_element_type=jnp.float32)
    y_ref[...] = y + b4_ref[...]


def _proj_kernel(q_ref, xt_ref, x_ref, isig_ref, out_ref):
    q = q_ref[0]
    pt = xt_ref[0]
    p = x_ref[0]
    inv_sigma = isig_ref[0, 0]

    q2 = jnp.sum(q * q, axis=1, keepdims=True)
    qa = jnp.concatenate([-2.0 * q, q2], axis=1)
    pt4 = jnp.concatenate([pt, jnp.ones((1, N), jnp.float32)], axis=0)
    qp2 = jnp.dot(qa, pt4, preferred_element_type=jnp.float32)
    p2 = jnp.sum(pt * pt, axis=0, keepdims=True)
    d2sel = qp2 + p2

    dmin = jnp.min(d2sel, axis=1, keepdims=True)
    t = dmin
    for _ in range(K - 1):
        t = jnp.min(jnp.where(d2sel <= t, jnp.inf, d2sel), axis=1,
                    keepdims=True)

    c2 = inv_sigma * 1.4426950408889634
    w = jnp.where(d2sel <= t, jnp.exp2((dmin - d2sel) * c2), 0.0)

    p4 = jnp.concatenate([p, jnp.ones((N, 1), jnp.float32)], axis=1)
    res = jnp.dot(w, p4, preferred_element_type=jnp.float32)
    out_ref[0] = res[:, 0:3] / res[:, 3:4]


def _run_mlp(global_feat, W1t, b1, g1, be1, W2t, b2, g2, be2, W3t, b3, g3,
             be3, W4t, b4):
    return pl.pallas_call(
        _mlp_kernel,
        out_shape=jax.ShapeDtypeStruct((B, 3 * M), jnp.float32),
    )(global_feat, W1t, b1, g1, be1, W2t, b2, g2, be2, W3t, b3, g3, be3,
      W4t, b4)


def _run_proj(generated, xt, x, inv_sigma, nb):
    return pl.pallas_call(
        _proj_kernel,
        grid=(nb,),
        in_specs=[
            pl.BlockSpec((1, M, 3), lambda b: (b, 0, 0)),
            pl.BlockSpec((1, 3, N), lambda b: (b, 0, 0)),
            pl.BlockSpec((1, N, 3), lambda b: (b, 0, 0)),
            pl.BlockSpec((1, 1), lambda b: (0, 0)),
        ],
        out_specs=pl.BlockSpec((1, M, 3), lambda b: (b, 0, 0)),
        out_shape=jax.ShapeDtypeStruct((nb, M, 3), jnp.float32),
    )(generated, xt, x, inv_sigma)


def kernel(x, global_feat, W1, b1, g1, be1, W2, b2, g2, be2, W3, b3, g3, be3,
           W4, b4, temperature):
    f32 = jnp.float32
    sigma = jnp.maximum(temperature * temperature, 0.01)
    inv_sigma = (1.0 / sigma).reshape(1, 1).astype(f32)
    mlp_args = (global_feat, W1.T, b1.reshape(1, -1), g1.reshape(1, -1),
                be1.reshape(1, -1), W2.T, b2.reshape(1, -1),
                g2.reshape(1, -1), be2.reshape(1, -1), W3.T,
                b3.reshape(1, -1), g3.reshape(1, -1), be3.reshape(1, -1),
                W4.T, b4.reshape(1, -1))

    y = _run_mlp(*mlp_args)
    generated = jnp.transpose(y.reshape(B, 3, M), (0, 2, 1))
    proj = _run_proj(generated, jnp.transpose(x, (0, 2, 1)), x, inv_sigma, B)
    return generated, proj


# speedup vs baseline: 1.9714x; 1.0671x over previous
import jax
import jax.numpy as jnp
from jax.experimental import pallas as pl


B, N, M, K = 32, 2048, 512, 8
BOTTLENECK = 1024


def _mlp_kernel(gf_ref, w1_ref, b1_ref, g1_ref, be1_ref,
                w2_ref, b2_ref, g2_ref, be2_ref,
                w3_ref, b3_ref, g3_ref, be3_ref,
                w4_ref, b4_ref, y_ref):
    def bn_relu(y, g, be):
        mean = jnp.mean(y, axis=0, keepdims=True)
        var = jnp.mean((y - mean) * (y - mean), axis=0, keepdims=True)
        return jax.nn.relu((y - mean) * jax.lax.rsqrt(var + 1e-5) * g + be)

    y = jnp.dot(gf_ref[...], w1_ref[...], preferred_element_type=jnp.float32)
    y = bn_relu(y + b1_ref[...], g1_ref[...], be1_ref[...])
    y = jnp.dot(y, w2_ref[...], preferred_element_type=jnp.float32)
    y = bn_relu(y + b2_ref[...], g2_ref[...], be2_ref[...])
    y = jnp.dot(y, w3_ref[...], preferred_element_type=jnp.float32)
    y = bn_relu(y + b3_ref[...], g3_ref[...], be3_ref[...])
    y = jnp.dot(y, w4_ref[...], preferred_element_type=jnp.float32)
    y_ref[...] = y + b4_ref[...]


def _proj_kernel(q_ref, xt_ref, isig_ref, out_ref):
    q = q_ref[0]
    pt = xt_ref[0]
    inv_sigma = isig_ref[0, 0]

    q2 = jnp.sum(q * q, axis=1, keepdims=True)
    qa = jnp.concatenate([-2.0 * q, q2], axis=1)
    pt4 = jnp.concatenate([pt, jnp.ones((1, N), jnp.float32)], axis=0)
    qp2 = jnp.dot(qa, pt4, preferred_element_type=jnp.float32)
    p2 = jnp.sum(pt * pt, axis=0, keepdims=True)
    d2sel = qp2 + p2

    dmin = jnp.min(d2sel, axis=1, keepdims=True)
    t = dmin
    for _ in range(K - 1):
        t = jnp.min(jnp.where(d2sel <= t, jnp.inf, d2sel), axis=1,
                    keepdims=True)

    c2 = inv_sigma * 1.4426950408889634
    w = jnp.where(d2sel <= t, jnp.exp2((dmin - d2sel) * c2), 0.0)

    px = pt[0:1, :]
    py = pt[1:2, :]
    pz = pt[2:3, :]
    wsum = jnp.sum(w, axis=1, keepdims=True)
    ox = jnp.sum(w * px, axis=1, keepdims=True)
    oy = jnp.sum(w * py, axis=1, keepdims=True)
    oz = jnp.sum(w * pz, axis=1, keepdims=True)
    out_ref[0] = jnp.concatenate([ox, oy, oz], axis=1) / wsum


def _run_mlp(global_feat, W1t, b1, g1, be1, W2t, b2, g2, be2, W3t, b3, g3,
             be3, W4t, b4):
    return pl.pallas_call(
        _mlp_kernel,
        out_shape=jax.ShapeDtypeStruct((B, 3 * M), jnp.float32),
    )(global_feat, W1t, b1, g1, be1, W2t, b2, g2, be2, W3t, b3, g3, be3,
      W4t, b4)


def _run_proj(generated, xt, inv_sigma, nb):
    return pl.pallas_call(
        _proj_kernel,
        grid=(nb,),
        in_specs=[
            pl.BlockSpec((1, M, 3), lambda b: (b, 0, 0)),
            pl.BlockSpec((1, 3, N), lambda b: (b, 0, 0)),
            pl.BlockSpec((1, 1), lambda b: (0, 0)),
        ],
        out_specs=pl.BlockSpec((1, M, 3), lambda b: (b, 0, 0)),
        out_shape=jax.ShapeDtypeStruct((nb, M, 3), jnp.float32),
    )(generated, xt, inv_sigma)


def kernel(x, global_feat, W1, b1, g1, be1, W2, b2, g2, be2, W3, b3, g3, be3,
           W4, b4, temperature):
    f32 = jnp.float32
    sigma = jnp.maximum(temperature * temperature, 0.01)
    inv_sigma = (1.0 / sigma).reshape(1, 1).astype(f32)
    mlp_args = (global_feat, W1.T, b1.reshape(1, -1), g1.reshape(1, -1),
                be1.reshape(1, -1), W2.T, b2.reshape(1, -1),
                g2.reshape(1, -1), be2.reshape(1, -1), W3.T,
                b3.reshape(1, -1), g3.reshape(1, -1), be3.reshape(1, -1),
                W4.T, b4.reshape(1, -1))

    y = _run_mlp(*mlp_args)
    generated = jnp.transpose(y.reshape(B, 3, M), (0, 2, 1))
    proj = _run_proj(generated, jnp.transpose(x, (0, 2, 1)), inv_sigma, B)
    return generated, proj


# speedup vs baseline: 2.2249x; 1.1286x over previous
import jax
import jax.numpy as jnp
from jax.experimental import pallas as pl


B, N, M, K = 32, 2048, 512, 8
BOTTLENECK = 1024


def _mlp_kernel(gf_ref, w1_ref, b1_ref, g1_ref, be1_ref,
                w2_ref, b2_ref, g2_ref, be2_ref,
                w3_ref, b3_ref, g3_ref, be3_ref,
                w4_ref, b4_ref, y_ref):
    def bn_relu(y, g, be):
        mean = jnp.mean(y, axis=0, keepdims=True)
        var = jnp.mean((y - mean) * (y - mean), axis=0, keepdims=True)
        return jax.nn.relu((y - mean) * jax.lax.rsqrt(var + 1e-5) * g + be)

    y = jnp.dot(gf_ref[...], w1_ref[...], preferred_element_type=jnp.float32)
    y = bn_relu(y + b1_ref[...], g1_ref[...], be1_ref[...])
    y = jnp.dot(y, w2_ref[...], preferred_element_type=jnp.float32)
    y = bn_relu(y + b2_ref[...], g2_ref[...], be2_ref[...])
    y = jnp.dot(y, w3_ref[...], preferred_element_type=jnp.float32)
    y = bn_relu(y + b3_ref[...], g3_ref[...], be3_ref[...])
    y = jnp.dot(y, w4_ref[...], preferred_element_type=jnp.float32)
    y_ref[...] = y + b4_ref[...]


def _merge_sorted(a, b):
    n = len(a)
    s = a + b[::-1]
    d = n
    while d >= 1:
        for i0 in range(0, 2 * n, 2 * d):
            for i in range(i0, i0 + d):
                lo = jnp.minimum(s[i], s[i + d])
                hi = jnp.maximum(s[i], s[i + d])
                s[i], s[i + d] = lo, hi
        d //= 2
    return s


def _proj_kernel(q_ref, xt_ref, isig_ref, out_ref):
    q = q_ref[0]
    pt = xt_ref[0]
    inv_sigma = isig_ref[0, 0]

    qp2 = jnp.dot(-2.0 * q, pt, preferred_element_type=jnp.float32)
    q2 = jnp.sum(q * q, axis=1, keepdims=True)
    p2 = jnp.sum(pt * pt, axis=0, keepdims=True)
    d2sel = (q2 + qp2) + p2

    cw = N // 16
    c = [d2sel[:, j * cw:(j + 1) * cw] for j in range(16)]
    pairs = [_merge_sorted([c[2 * j]], [c[2 * j + 1]]) for j in range(8)]
    quads = [_merge_sorted(pairs[2 * j], pairs[2 * j + 1]) for j in range(4)]
    octs = [_merge_sorted(quads[0], quads[1]),
            _merge_sorted(quads[2], quads[3])]
    cand = [jnp.minimum(octs[0][i], octs[1][7 - i]) for i in range(8)]

    m = cand[0]
    for cd in cand[1:]:
        m = jnp.minimum(m, cd)
    dmin = jnp.min(m, axis=1, keepdims=True)
    t = dmin
    for _ in range(K - 1):
        mm = None
        for cd in cand:
            x = jnp.where(cd <= t, jnp.inf, cd)
            mm = x if mm is None else jnp.minimum(mm, x)
        t = jnp.min(mm, axis=1, keepdims=True)

    c2 = inv_sigma * 1.4426950408889634
    w = jnp.where(d2sel <= t, jnp.exp2((dmin - d2sel) * c2), 0.0)

    px = pt[0:1, :]
    py = pt[1:2, :]
    pz = pt[2:3, :]
    wsum = jnp.sum(w, axis=1, keepdims=True)
    ox = jnp.sum(w * px, axis=1, keepdims=True)
    oy = jnp.sum(w * py, axis=1, keepdims=True)
    oz = jnp.sum(w * pz, axis=1, keepdims=True)
    out_ref[0] = jnp.concatenate([ox, oy, oz], axis=1) / wsum


def _run_mlp(global_feat, W1t, b1, g1, be1, W2t, b2, g2, be2, W3t, b3, g3,
             be3, W4t, b4):
    return pl.pallas_call(
        _mlp_kernel,
        out_shape=jax.ShapeDtypeStruct((B, 3 * M), jnp.float32),
    )(global_feat, W1t, b1, g1, be1, W2t, b2, g2, be2, W3t, b3, g3, be3,
      W4t, b4)


def _run_proj(generated, xt, inv_sigma, nb):
    return pl.pallas_call(
        _proj_kernel,
        grid=(nb,),
        in_specs=[
            pl.BlockSpec((1, M, 3), lambda b: (b, 0, 0)),
            pl.BlockSpec((1, 3, N), lambda b: (b, 0, 0)),
            pl.BlockSpec((1, 1), lambda b: (0, 0)),
        ],
        out_specs=pl.BlockSpec((1, M, 3), lambda b: (b, 0, 0)),
        out_shape=jax.ShapeDtypeStruct((nb, M, 3), jnp.float32),
    )(generated, xt, inv_sigma)


def kernel(x, global_feat, W1, b1, g1, be1, W2, b2, g2, be2, W3, b3, g3, be3,
           W4, b4, temperature):
    f32 = jnp.float32
    sigma = jnp.maximum(temperature * temperature, 0.01)
    inv_sigma = (1.0 / sigma).reshape(1, 1).astype(f32)
    mlp_args = (global_feat, W1.T, b1.reshape(1, -1), g1.reshape(1, -1),
                be1.reshape(1, -1), W2.T, b2.reshape(1, -1),
                g2.reshape(1, -1), be2.reshape(1, -1), W3.T,
                b3.reshape(1, -1), g3.reshape(1, -1), be3.reshape(1, -1),
                W4.T, b4.reshape(1, -1))

    y = _run_mlp(*mlp_args)
    generated = jnp.transpose(y.reshape(B, 3, M), (0, 2, 1))
    proj = _run_proj(generated, jnp.transpose(x, (0, 2, 1)), inv_sigma, B)
    return generated, proj


# speedup vs baseline: 2.4269x; 1.0908x over previous
import jax
import jax.numpy as jnp
from jax.experimental import pallas as pl


B, N, M, K = 32, 2048, 512, 8
BOTTLENECK = 1024


def _mlp_kernel(gf_ref, w1_ref, b1_ref, g1_ref, be1_ref,
                w2_ref, b2_ref, g2_ref, be2_ref,
                w3_ref, b3_ref, g3_ref, be3_ref,
                w4_ref, b4_ref, y_ref):
    def bn_relu(y, g, be):
        mean = jnp.mean(y, axis=0, keepdims=True)
        var = jnp.mean((y - mean) * (y - mean), axis=0, keepdims=True)
        return jax.nn.relu((y - mean) * jax.lax.rsqrt(var + 1e-5) * g + be)

    y = jnp.dot(gf_ref[...], w1_ref[...], preferred_element_type=jnp.float32)
    y = bn_relu(y + b1_ref[...], g1_ref[...], be1_ref[...])
    y = jnp.dot(y, w2_ref[...], preferred_element_type=jnp.float32)
    y = bn_relu(y + b2_ref[...], g2_ref[...], be2_ref[...])
    y = jnp.dot(y, w3_ref[...], preferred_element_type=jnp.float32)
    y = bn_relu(y + b3_ref[...], g3_ref[...], be3_ref[...])
    y = jnp.dot(y, w4_ref[...], preferred_element_type=jnp.float32)
    y_ref[...] = y + b4_ref[...]


def _merge_sorted(a, b):
    n = len(a)
    s = a + b[::-1]
    d = n
    while d >= 1:
        for i0 in range(0, 2 * n, 2 * d):
            for i in range(i0, i0 + d):
                lo = jnp.minimum(s[i], s[i + d])
                hi = jnp.maximum(s[i], s[i + d])
                s[i], s[i + d] = lo, hi
        d //= 2
    return s


def _bitonic_sort8(s):
    s = list(s)
    d = 4
    while d >= 1:
        for i0 in range(0, 8, 2 * d):
            for i in range(i0, i0 + d):
                lo = jnp.minimum(s[i], s[i + d])
                hi = jnp.maximum(s[i], s[i + d])
                s[i], s[i + d] = lo, hi
        d //= 2
    return s


def _proj_kernel(q_ref, xt_ref, isig_ref, out_ref):
    q = q_ref[0]
    pt = xt_ref[0]
    inv_sigma = isig_ref[0, 0]

    qp2 = jnp.dot(-2.0 * q, pt, preferred_element_type=jnp.float32)
    q2 = jnp.sum(q * q, axis=1, keepdims=True)
    p2 = jnp.sum(pt * pt, axis=0, keepdims=True)
    d2sel = (q2 + qp2) + p2

    cw = N // 16
    c = [d2sel[:, j * cw:(j + 1) * cw] for j in range(16)]
    pairs = [_merge_sorted([c[2 * j]], [c[2 * j + 1]]) for j in range(8)]
    quads = [_merge_sorted(pairs[2 * j], pairs[2 * j + 1]) for j in range(4)]
    octs = [_merge_sorted(quads[0], quads[1]),
            _merge_sorted(quads[2], quads[3])]
    low = [jnp.minimum(octs[0][i], octs[1][7 - i]) for i in range(8)]
    cand = _bitonic_sort8(low)

    dmin = jnp.min(cand[0], axis=1, keepdims=True)
    t = dmin
    for i in range(2, K + 1):
        mm = None
        for cd in cand[:i]:
            x = jnp.where(cd <= t, jnp.inf, cd)
            mm = x if mm is None else jnp.minimum(mm, x)
        t = jnp.min(mm, axis=1, keepdims=True)

    nc2 = inv_sigma * (-1.4426950408889634)
    w = jnp.where(d2sel <= t, jnp.exp2(d2sel * nc2), 0.0)

    px = pt[0:1, :]
    py = pt[1:2, :]
    pz = pt[2:3, :]
    wsum = jnp.sum(w, axis=1, keepdims=True)
    ox = jnp.sum(w * px, axis=1, keepdims=True)
    oy = jnp.sum(w * py, axis=1, keepdims=True)
    oz = jnp.sum(w * pz, axis=1, keepdims=True)
    out_ref[0] = jnp.concatenate([ox, oy, oz], axis=1) / wsum


def _run_mlp(global_feat, W1t, b1, g1, be1, W2t, b2, g2, be2, W3t, b3, g3,
             be3, W4t, b4):
    return pl.pallas_call(
        _mlp_kernel,
        out_shape=jax.ShapeDtypeStruct((B, 3 * M), jnp.float32),
    )(global_feat, W1t, b1, g1, be1, W2t, b2, g2, be2, W3t, b3, g3, be3,
      W4t, b4)


def _run_proj(generated, xt, inv_sigma, nb):
    return pl.pallas_call(
        _proj_kernel,
        grid=(nb,),
        in_specs=[
            pl.BlockSpec((1, M, 3), lambda b: (b, 0, 0)),
            pl.BlockSpec((1, 3, N), lambda b: (b, 0, 0)),
            pl.BlockSpec((1, 1), lambda b: (0, 0)),
        ],
        out_specs=pl.BlockSpec((1, M, 3), lambda b: (b, 0, 0)),
        out_shape=jax.ShapeDtypeStruct((nb, M, 3), jnp.float32),
    )(generated, xt, inv_sigma)


def kernel(x, global_feat, W1, b1, g1, be1, W2, b2, g2, be2, W3, b3, g3, be3,
           W4, b4, temperature):
    f32 = jnp.float32
    sigma = jnp.maximum(temperature * temperature, 0.01)
    inv_sigma = (1.0 / sigma).reshape(1, 1).astype(f32)
    mlp_args = (global_feat, W1.T, b1.reshape(1, -1), g1.reshape(1, -1),
                be1.reshape(1, -1), W2.T, b2.reshape(1, -1),
                g2.reshape(1, -1), be2.reshape(1, -1), W3.T,
                b3.reshape(1, -1), g3.reshape(1, -1), be3.reshape(1, -1),
                W4.T, b4.reshape(1, -1))

    y = _run_mlp(*mlp_args)
    generated = jnp.transpose(y.reshape(B, 3, M), (0, 2, 1))
    proj = _run_proj(generated, jnp.transpose(x, (0, 2, 1)), inv_sigma, B)
    return generated, proj


# speedup vs baseline: 2.4750x; 1.0198x over previous
import jax
import jax.numpy as jnp
from jax.experimental import pallas as pl


B, N, M, K = 32, 2048, 512, 8
BOTTLENECK = 1024


def _mlp_kernel(gf_ref, w1_ref, b1_ref, g1_ref, be1_ref,
                w2_ref, b2_ref, g2_ref, be2_ref,
                w3_ref, b3_ref, g3_ref, be3_ref,
                w4_ref, b4_ref, y_ref):
    def bn_relu(y, g, be):
        mean = jnp.mean(y, axis=0, keepdims=True)
        var = jnp.mean((y - mean) * (y - mean), axis=0, keepdims=True)
        return jax.nn.relu((y - mean) * jax.lax.rsqrt(var + 1e-5) * g + be)

    y = jnp.dot(gf_ref[...], w1_ref[...], preferred_element_type=jnp.float32)
    y = bn_relu(y + b1_ref[...], g1_ref[...], be1_ref[...])
    y = jnp.dot(y, w2_ref[...], preferred_element_type=jnp.float32)
    y = bn_relu(y + b2_ref[...], g2_ref[...], be2_ref[...])
    y = jnp.dot(y, w3_ref[...], preferred_element_type=jnp.float32)
    y = bn_relu(y + b3_ref[...], g3_ref[...], be3_ref[...])
    y = jnp.dot(y, w4_ref[...], preferred_element_type=jnp.float32)
    y_ref[...] = y + b4_ref[...]


def _merge_sorted(a, b):
    n = len(a)
    s = a + b[::-1]
    d = n
    while d >= 1:
        for i0 in range(0, 2 * n, 2 * d):
            for i in range(i0, i0 + d):
                lo = jnp.minimum(s[i], s[i + d])
                hi = jnp.maximum(s[i], s[i + d])
                s[i], s[i + d] = lo, hi
        d //= 2
    return s


def _bitonic_sort8(s):
    s = list(s)
    d = 4
    while d >= 1:
        for i0 in range(0, 8, 2 * d):
            for i in range(i0, i0 + d):
                lo = jnp.minimum(s[i], s[i + d])
                hi = jnp.maximum(s[i], s[i + d])
                s[i], s[i + d] = lo, hi
        d //= 2
    return s


def _proj_kernel(q_ref, xt_ref, isig_ref, out_ref):
    q = q_ref[0]
    pt = xt_ref[0]
    inv_sigma = isig_ref[0, 0]

    nc2 = inv_sigma * (-1.4426950408889634)
    qs = (-2.0 * nc2) * q
    qp2 = jnp.dot(qs, pt, preferred_element_type=jnp.float32)
    q2 = nc2 * jnp.sum(q * q, axis=1, keepdims=True)
    p2 = nc2 * jnp.sum(pt * pt, axis=0, keepdims=True)
    ds = (q2 + qp2) + p2

    cw = N // 16
    c = [ds[:, j * cw:(j + 1) * cw] for j in range(16)]
    pairs = [_merge_sorted([c[2 * j]], [c[2 * j + 1]]) for j in range(8)]
    quads = [_merge_sorted(pairs[2 * j], pairs[2 * j + 1]) for j in range(4)]
    octs = [_merge_sorted(quads[0], quads[1]),
            _merge_sorted(quads[2], quads[3])]
    up = [jnp.maximum(octs[0][i], octs[1][7 - i]) for i in range(8)]
    cand = _bitonic_sort8(up)

    t = jnp.max(cand[7], axis=1, keepdims=True)
    for i in range(2, K + 1):
        mm = None
        for cd in cand[8 - i:]:
            x = jnp.where(cd >= t, -jnp.inf, cd)
            mm = x if mm is None else jnp.maximum(mm, x)
        t = jnp.max(mm, axis=1, keepdims=True)

    w = jnp.where(ds >= t, jnp.exp2(ds), 0.0)

    px = pt[0:1, :]
    py = pt[1:2, :]
    pz = pt[2:3, :]
    wsum = jnp.sum(w, axis=1, keepdims=True)
    ox = jnp.sum(w * px, axis=1, keepdims=True)
    oy = jnp.sum(w * py, axis=1, keepdims=True)
    oz = jnp.sum(w * pz, axis=1, keepdims=True)
    out_ref[0] = jnp.concatenate([ox, oy, oz], axis=1) / wsum


def _run_mlp(global_feat, W1t, b1, g1, be1, W2t, b2, g2, be2, W3t, b3, g3,
             be3, W4t, b4):
    return pl.pallas_call(
        _mlp_kernel,
        out_shape=jax.ShapeDtypeStruct((B, 3 * M), jnp.float32),
    )(global_feat, W1t, b1, g1, be1, W2t, b2, g2, be2, W3t, b3, g3, be3,
      W4t, b4)


def _run_proj(generated, xt, inv_sigma, nb):
    return pl.pallas_call(
        _proj_kernel,
        grid=(nb,),
        in_specs=[
            pl.BlockSpec((1, M, 3), lambda b: (b, 0, 0)),
            pl.BlockSpec((1, 3, N), lambda b: (b, 0, 0)),
            pl.BlockSpec((1, 1), lambda b: (0, 0)),
        ],
        out_specs=pl.BlockSpec((1, M, 3), lambda b: (b, 0, 0)),
        out_shape=jax.ShapeDtypeStruct((nb, M, 3), jnp.float32),
    )(generated, xt, inv_sigma)


def kernel(x, global_feat, W1, b1, g1, be1, W2, b2, g2, be2, W3, b3, g3, be3,
           W4, b4, temperature):
    f32 = jnp.float32
    sigma = jnp.maximum(temperature * temperature, 0.01)
    inv_sigma = (1.0 / sigma).reshape(1, 1).astype(f32)
    mlp_args = (global_feat, W1.T, b1.reshape(1, -1), g1.reshape(1, -1),
                be1.reshape(1, -1), W2.T, b2.reshape(1, -1),
                g2.reshape(1, -1), be2.reshape(1, -1), W3.T,
                b3.reshape(1, -1), g3.reshape(1, -1), be3.reshape(1, -1),
                W4.T, b4.reshape(1, -1))

    y = _run_mlp(*mlp_args)
    generated = jnp.transpose(y.reshape(B, 3, M), (0, 2, 1))
    proj = _run_proj(generated, jnp.transpose(x, (0, 2, 1)), inv_sigma, B)
    return generated, proj


# speedup vs baseline: 2.4787x; 1.0015x over previous
import jax
import jax.numpy as jnp
from jax.experimental import pallas as pl


B, N, M, K = 32, 2048, 512, 8
BOTTLENECK = 1024


def _mlp_kernel(gf_ref, w1_ref, b1_ref, g1_ref, be1_ref,
                w2_ref, b2_ref, g2_ref, be2_ref,
                w3_ref, b3_ref, g3_ref, be3_ref,
                w4_ref, b4_ref, y_ref):
    def bn_relu(y, g, be):
        mean = jnp.mean(y, axis=0, keepdims=True)
        var = jnp.mean((y - mean) * (y - mean), axis=0, keepdims=True)
        return jax.nn.relu((y - mean) * jax.lax.rsqrt(var + 1e-5) * g + be)

    y = jnp.dot(gf_ref[...], w1_ref[...], preferred_element_type=jnp.float32)
    y = bn_relu(y + b1_ref[...], g1_ref[...], be1_ref[...])
    y = jnp.dot(y, w2_ref[...], preferred_element_type=jnp.float32)
    y = bn_relu(y + b2_ref[...], g2_ref[...], be2_ref[...])
    y = jnp.dot(y, w3_ref[...], preferred_element_type=jnp.float32)
    y = bn_relu(y + b3_ref[...], g3_ref[...], be3_ref[...])
    y = jnp.dot(y, w4_ref[...], preferred_element_type=jnp.float32)
    y_ref[...] = y + b4_ref[...]


def _merge_sorted(a, b):
    n = len(a)
    s = a + b[::-1]
    d = n
    while d >= 1:
        for i0 in range(0, 2 * n, 2 * d):
            for i in range(i0, i0 + d):
                lo = jnp.minimum(s[i], s[i + d])
                hi = jnp.maximum(s[i], s[i + d])
                s[i], s[i + d] = lo, hi
        d //= 2
    return s


def _bitonic_sort8(s):
    s = list(s)
    d = 4
    while d >= 1:
        for i0 in range(0, 8, 2 * d):
            for i in range(i0, i0 + d):
                lo = jnp.minimum(s[i], s[i + d])
                hi = jnp.maximum(s[i], s[i + d])
                s[i], s[i + d] = lo, hi
        d //= 2
    return s


def _proj_kernel(q_ref, xt_ref, isig_ref, out_ref):
    inv_sigma = isig_ref[0, 0]
    for s in range(q_ref.shape[0]):
        out_ref[s] = _soft_proj(q_ref[s], xt_ref[s], inv_sigma)


def _soft_proj(q, pt, inv_sigma):
    qp2 = jnp.dot(-2.0 * q, pt, preferred_element_type=jnp.float32)
    q2 = jnp.sum(q * q, axis=1, keepdims=True)
    p2 = jnp.sum(pt * pt, axis=0, keepdims=True)
    d2sel = (q2 + qp2) + p2

    cw = N // 16
    c = [d2sel[:, j * cw:(j + 1) * cw] for j in range(16)]
    pairs = [_merge_sorted([c[2 * j]], [c[2 * j + 1]]) for j in range(8)]
    quads = [_merge_sorted(pairs[2 * j], pairs[2 * j + 1]) for j in range(4)]
    octs = [_merge_sorted(quads[0], quads[1]),
            _merge_sorted(quads[2], quads[3])]
    low = [jnp.minimum(octs[0][i], octs[1][7 - i]) for i in range(8)]
    cand = _bitonic_sort8(low)

    t = jnp.min(cand[0], axis=1, keepdims=True)
    for i in range(2, K + 1):
        mm = None
        for cd in cand[:i]:
            x = jnp.where(cd <= t, jnp.inf, cd)
            mm = x if mm is None else jnp.minimum(mm, x)
        t = jnp.min(mm, axis=1, keepdims=True)

    nc2 = inv_sigma * (-1.4426950408889634)
    w = jnp.where(d2sel <= t, jnp.exp2(d2sel * nc2), 0.0)

    px = pt[0:1, :]
    py = pt[1:2, :]
    pz = pt[2:3, :]
    wsum = jnp.sum(w, axis=1, keepdims=True)
    ox = jnp.sum(w * px, axis=1, keepdims=True)
    oy = jnp.sum(w * py, axis=1, keepdims=True)
    oz = jnp.sum(w * pz, axis=1, keepdims=True)
    return jnp.concatenate([ox, oy, oz], axis=1) / wsum


def _run_mlp(global_feat, W1t, b1, g1, be1, W2t, b2, g2, be2, W3t, b3, g3,
             be3, W4t, b4):
    return pl.pallas_call(
        _mlp_kernel,
        out_shape=jax.ShapeDtypeStruct((B, 3 * M), jnp.float32),
    )(global_feat, W1t, b1, g1, be1, W2t, b2, g2, be2, W3t, b3, g3, be3,
      W4t, b4)


def _run_proj(generated, xt, inv_sigma, nb, bps=2):
    return pl.pallas_call(
        _proj_kernel,
        grid=(nb // bps,),
        in_specs=[
            pl.BlockSpec((bps, M, 3), lambda b: (b, 0, 0)),
            pl.BlockSpec((bps, 3, N), lambda b: (b, 0, 0)),
            pl.BlockSpec((1, 1), lambda b: (0, 0)),
        ],
        out_specs=pl.BlockSpec((bps, M, 3), lambda b: (b, 0, 0)),
        out_shape=jax.ShapeDtypeStruct((nb, M, 3), jnp.float32),
    )(generated, xt, inv_sigma)


def kernel(x, global_feat, W1, b1, g1, be1, W2, b2, g2, be2, W3, b3, g3, be3,
           W4, b4, temperature):
    f32 = jnp.float32
    sigma = jnp.maximum(temperature * temperature, 0.01)
    inv_sigma = (1.0 / sigma).reshape(1, 1).astype(f32)
    mlp_args = (global_feat, W1.T, b1.reshape(1, -1), g1.reshape(1, -1),
                be1.reshape(1, -1), W2.T, b2.reshape(1, -1),
                g2.reshape(1, -1), be2.reshape(1, -1), W3.T,
                b3.reshape(1, -1), g3.reshape(1, -1), be3.reshape(1, -1),
                W4.T, b4.reshape(1, -1))

    y = _run_mlp(*mlp_args)
    generated = jnp.transpose(y.reshape(B, 3, M), (0, 2, 1))
    proj = _run_proj(generated, jnp.transpose(x, (0, 2, 1)), inv_sigma, B)
    return generated, proj
